# baseline (speedup 1.0000x reference)
import functools

import jax
import jax.numpy as jnp
from jax import lax
from jax.experimental import pallas as pl
from jax.experimental.pallas import tpu as pltpu
from jax.experimental.pallas import tpu_sc as plsc

_R, _N, _S = 4, 50000, 10
_B, _F, _H = 512, 256, 256
_M = _B * _S
_RM = _R * _M
_NC, _NS = 2, 16
_NW = _NC * _NS
_PER_W = _RM // _NW
_CH = 8
_NCH = _PER_W // _CH
_NB = 4
_IPS = _CH * _S
_IDXW = _PER_W * _S


def _sc_gather_sum(emb, idx1):
    mesh = plsc.VectorSubcoreMesh(core_axis_name="c", subcore_axis_name="s")

    @functools.partial(
        pl.kernel,
        out_type=jax.ShapeDtypeStruct((_RM, _F), jnp.float32),
        mesh=mesh,
        scratch_types=[
            pltpu.VMEM((_IDXW,), jnp.int32),
            pltpu.VMEM((_NB, _CH * _S, _F), jnp.float32),
            pltpu.VMEM((_NB, _CH, _F), jnp.float32),
        ]
        + [pltpu.SemaphoreType.DMA] * (2 * _NB),
    )
    def k(emb_hbm, idx_hbm, out_hbm, idx_all, rows_v, out_v, *sems):
        wid = lax.axis_index("s") * _NC + lax.axis_index("c")
        base = wid * _PER_W
        gsem = sems[:_NB]
        osem = sems[_NB:]

        pltpu.sync_copy(idx_hbm.at[pl.ds(wid * _IDXW, _IDXW)], idx_all)

        def gather_cp(g, slot):
            return pltpu.make_async_copy(
                emb_hbm.at[idx_all.at[pl.ds(g * _IPS, _IPS)]],
                rows_v.at[slot],
                gsem[slot],
            )

        def out_cp(g, slot):
            return pltpu.make_async_copy(
                out_v.at[slot], out_hbm.at[pl.ds(base + g * _CH, _CH)], osem[slot]
            )

        def compute(g, slot):
            @pl.when(g >= _NB)
            def _drain():
                out_cp(g - _NB, slot).wait()

            def row(i, c2):
                for f in range(_F // 16):
                    sl = pl.ds(f * 16, 16)
                    acc = rows_v[slot, i * _S, sl]
                    for s2 in range(1, _S):
                        acc = acc + rows_v[slot, i * _S + s2, sl]
                    out_v[slot, i, sl] = acc
                return c2

            lax.fori_loop(0, _CH, row, 0)
            out_cp(g, slot).start()

        for p in range(_NB - 1):
            gather_cp(p, p).start()

        def bodyn(h, carry):
            g = _NB * h
            for slot in range(_NB):
                gg = g + slot
                gather_cp(gg, slot).wait()

                @pl.when(gg + _NB - 1 < _NCH)
                def _next():
                    gather_cp(gg + _NB - 1, (slot + _NB - 1) % _NB).start()

                compute(gg, slot)
            return carry

        lax.fori_loop(0, _NCH // _NB, bodyn, 0)
        for p in range(_NB):
            out_cp(_NCH - _NB + p, (_NCH - _NB + p) % _NB).wait()

    return k(emb, idx1)


def _tc_transform(agg, w1, w2):
    def body(a_ref, w1_ref, w2_ref, o_ref):
        r = pl.program_id(0)
        e = jnp.maximum(
            jnp.dot(a_ref[0], w1_ref[0], preferred_element_type=jnp.float32), 0.0
        )
        x = e.reshape(_B, _S, _H).sum(axis=1)
        h = jnp.maximum(
            jnp.dot(x, w2_ref[0], preferred_element_type=jnp.float32), 0.0
        )

        @pl.when(r == 0)
        def _init():
            o_ref[...] = h

        @pl.when(r != 0)
        def _acc():
            o_ref[...] += h

    return pl.pallas_call(
        body,
        grid=(_R,),
        in_specs=[
            pl.BlockSpec((1, _M, _F), lambda r: (r, 0, 0)),
            pl.BlockSpec((1, _F, _H), lambda r: (r, 0, 0)),
            pl.BlockSpec((1, _H, _H), lambda r: (r, 0, 0)),
        ],
        out_specs=pl.BlockSpec((_B, _H), lambda r: (0, 0)),
        out_shape=jax.ShapeDtypeStruct((_B, _H), jnp.float32),
    )(agg, w1, w2)


def kernel(emb, W1, W2, neigh, nodes):
    neighf = neigh.reshape(_R * _N, _S)
    rows1 = jnp.arange(_R, dtype=jnp.int32)[:, None] * _N + nodes[None, :]
    nb2 = neighf[rows1]
    rows2 = (jnp.arange(_R, dtype=jnp.int32)[:, None] * _N
             + nb2.reshape(_R, _B * _S))
    h1 = neighf[rows2]
    idx1 = h1.reshape(_RM * _S)
    agg = _sc_gather_sum(emb, idx1)
    inv_s = jnp.float32(1.0 / _S)
    return _tc_transform(agg.reshape(_R, _M, _F), W1 * inv_s, W2 * inv_s)


# speedup vs baseline: 1.0123x; 1.0123x over previous
import functools

import jax
import jax.numpy as jnp
from jax import lax
from jax.experimental import pallas as pl
from jax.experimental.pallas import tpu as pltpu
from jax.experimental.pallas import tpu_sc as plsc

_R, _N, _S = 4, 50000, 10
_B, _F, _H = 512, 256, 256
_M = _B * _S
_RM = _R * _M
_NC, _NS = 2, 16
_NW = _NC * _NS
_PER_W = _RM // _NW
_CH = 16
_NCH = _PER_W // _CH
_SPC = 2
_IPS = _CH * _S // _SPC
_RPS = _CH // _SPC
_IDXW = _PER_W * _S


def _sc_gather_sum(emb, idx1):
    mesh = plsc.VectorSubcoreMesh(core_axis_name="c", subcore_axis_name="s")

    @functools.partial(
        pl.kernel,
        out_type=jax.ShapeDtypeStruct((_RM, _F), jnp.float32),
        mesh=mesh,
        scratch_types=[
            pltpu.VMEM((_IDXW,), jnp.int32),
            pltpu.VMEM((2, _CH * _S, _F), jnp.float32),
            pltpu.VMEM((2, _CH, _F), jnp.float32),
        ]
        + [pltpu.SemaphoreType.DMA] * 6,
    )
    def k(emb_hbm, idx_hbm, out_hbm, idx_all, rows_v, out_v, *sems):
        wid = lax.axis_index("s") * _NC + lax.axis_index("c")
        base = wid * _PER_W
        gsem = (sems[0:2], sems[2:4])
        osem = sems[4:6]

        pltpu.sync_copy(idx_hbm.at[pl.ds(wid * _IDXW, _IDXW)], idx_all)

        def gather_cp(g, slot, j):
            return pltpu.make_async_copy(
                emb_hbm.at[idx_all.at[pl.ds(g * _CH * _S + j * _IPS, _IPS)]],
                rows_v.at[slot, pl.ds(j * _IPS, _IPS)],
                gsem[slot][j],
            )

        def out_cp(g, slot):
            return pltpu.make_async_copy(
                out_v.at[slot], out_hbm.at[pl.ds(base + g * _CH, _CH)], osem[slot]
            )

        def rows_half(slot, j):
            def row(i, c2):
                for f in range(_F // 16):
                    sl = pl.ds(f * 16, 16)
                    acc = rows_v[slot, i * _S, sl]
                    for s2 in range(1, _S):
                        acc = acc + rows_v[slot, i * _S + s2, sl]
                    out_v[slot, i, sl] = acc
                return c2

            lax.fori_loop(j * _RPS, (j + 1) * _RPS, row, 0)

        def issue(g, slot):
            for j in range(_SPC):
                gather_cp(g, slot, j).start()

        issue(0, 0)

        def body2(h, carry):
            g = 2 * h
            for slot in range(2):
                gg = g + slot

                @pl.when(gg >= 2)
                def _drain():
                    out_cp(gg - 2, slot).wait()

                for j in range(_SPC):
                    gather_cp(gg, slot, j).wait()

                    @pl.when(gg + 1 < _NCH)
                    def _next():
                        gather_cp(gg + 1, 1 - slot, j).start()

                    rows_half(slot, j)
                out_cp(gg, slot).start()
            return carry

        lax.fori_loop(0, _NCH // 2, body2, 0)
        out_cp(_NCH - 2, 0).wait()
        out_cp(_NCH - 1, 1).wait()

    return k(emb, idx1)


def _tc_transform(agg, w1, w2):
    def body(a_ref, w1_ref, w2_ref, o_ref):
        r = pl.program_id(0)
        e = jnp.maximum(
            jnp.dot(a_ref[0], w1_ref[0], preferred_element_type=jnp.float32), 0.0
        )
        x = e.reshape(_B, _S, _H).sum(axis=1)
        h = jnp.maximum(
            jnp.dot(x, w2_ref[0], preferred_element_type=jnp.float32), 0.0
        )

        @pl.when(r == 0)
        def _init():
            o_ref[...] = h

        @pl.when(r != 0)
        def _acc():
            o_ref[...] += h

    return pl.pallas_call(
        body,
        grid=(_R,),
        in_specs=[
            pl.BlockSpec((1, _M, _F), lambda r: (r, 0, 0)),
            pl.BlockSpec((1, _F, _H), lambda r: (r, 0, 0)),
            pl.BlockSpec((1, _H, _H), lambda r: (r, 0, 0)),
        ],
        out_specs=pl.BlockSpec((_B, _H), lambda r: (0, 0)),
        out_shape=jax.ShapeDtypeStruct((_B, _H), jnp.float32),
    )(agg, w1, w2)


def kernel(emb, W1, W2, neigh, nodes):
    neighf = neigh.reshape(_R * _N, _S)
    rows1 = jnp.arange(_R, dtype=jnp.int32)[:, None] * _N + nodes[None, :]
    nb2 = neighf[rows1]
    rows2 = (jnp.arange(_R, dtype=jnp.int32)[:, None] * _N
             + nb2.reshape(_R, _B * _S))
    h1 = neighf[rows2]
    idx1 = h1.reshape(_RM * _S)
    agg = _sc_gather_sum(emb, idx1)
    inv_s = jnp.float32(1.0 / _S)
    return _tc_transform(agg.reshape(_R, _M, _F), W1 * inv_s, W2 * inv_s)


# speedup vs baseline: 1.0921x; 1.0788x over previous
import functools

import jax
import jax.numpy as jnp
from jax import lax
from jax.experimental import pallas as pl
from jax.experimental.pallas import tpu as pltpu
from jax.experimental.pallas import tpu_sc as plsc

_R, _N, _S = 4, 50000, 10
_B, _F, _H = 512, 256, 256
_M = _B * _S
_RM = _R * _M
_NC, _NS = 2, 16
_NW = _NC * _NS
_PER_W = _RM // _NW
_CH = 16
_NCH = _PER_W // _CH
_SPC = 2
_IPS = _CH * _S // _SPC
_IDXW = _PER_W * _S


def _sc_gather_sum(emb, idx1):
    mesh = plsc.VectorSubcoreMesh(core_axis_name="c", subcore_axis_name="s")

    @functools.partial(
        pl.kernel,
        out_type=jax.ShapeDtypeStruct((_RM, _F), jnp.float32),
        mesh=mesh,
        scratch_types=[
            pltpu.VMEM((_IDXW,), jnp.int32),
            pltpu.VMEM((2, _CH * _S, _F), jnp.float32),
            pltpu.VMEM((2, _CH, _F), jnp.float32),
            pltpu.SemaphoreType.DMA,
            pltpu.SemaphoreType.DMA,
            pltpu.SemaphoreType.DMA,
            pltpu.SemaphoreType.DMA,
        ],
    )
    def k(emb_hbm, idx_hbm, out_hbm, idx_all, rows_v, out_v, g0, g1, o0, o1):
        wid = lax.axis_index("s") * _NC + lax.axis_index("c")
        base = wid * _PER_W
        gsem = (g0, g1)
        osem = (o0, o1)

        pltpu.sync_copy(idx_hbm.at[pl.ds(wid * _IDXW, _IDXW)], idx_all)

        def gather_cps(g, slot):
            return [
                pltpu.make_async_copy(
                    emb_hbm.at[idx_all.at[pl.ds(g * _CH * _S + j * _IPS, _IPS)]],
                    rows_v.at[slot, pl.ds(j * _IPS, _IPS)],
                    gsem[slot],
                )
                for j in range(_SPC)
            ]

        def out_cp(g, slot):
            return pltpu.make_async_copy(
                out_v.at[slot], out_hbm.at[pl.ds(base + g * _CH, _CH)], osem[slot]
            )

        def issue(g, slot):
            for cp in gather_cps(g, slot):
                cp.start()

        def compute(g, slot):
            @pl.when(g >= 2)
            def _drain():
                out_cp(g - 2, slot).wait()

            def row(i, c2):
                for f in range(_F // 16):
                    sl = pl.ds(f * 16, 16)
                    acc = rows_v[slot, i * _S, sl]
                    for s2 in range(1, _S):
                        acc = acc + rows_v[slot, i * _S + s2, sl]
                    out_v[slot, i, sl] = acc
                return c2

            lax.fori_loop(0, _CH, row, 0)
            out_cp(g, slot).start()

        issue(0, 0)

        def body2(h, carry):
            g = 2 * h
            for slot in range(2):
                gg = g + slot
                for cp in gather_cps(gg, slot):
                    cp.wait()

                @pl.when(gg + 1 < _NCH)
                def _next():
                    issue(gg + 1, 1 - slot)

                compute(gg, slot)
            return carry

        lax.fori_loop(0, _NCH // 2, body2, 0)
        out_cp(_NCH - 2, 0).wait()
        out_cp(_NCH - 1, 1).wait()

    return k(emb, idx1)


def _tc_transform(agg, w1, w2):
    def body(a_ref, w1_ref, w2_ref, o_ref):
        r = pl.program_id(0)
        e = jnp.maximum(
            jnp.dot(a_ref[0], w1_ref[0], preferred_element_type=jnp.float32), 0.0
        )
        x = e.reshape(_B, _S, _H).sum(axis=1)
        h = jnp.maximum(
            jnp.dot(x, w2_ref[0], preferred_element_type=jnp.float32), 0.0
        )

        @pl.when(r == 0)
        def _init():
            o_ref[...] = h

        @pl.when(r != 0)
        def _acc():
            o_ref[...] += h

    return pl.pallas_call(
        body,
        grid=(_R,),
        in_specs=[
            pl.BlockSpec((1, _M, _F), lambda r: (r, 0, 0)),
            pl.BlockSpec((1, _F, _H), lambda r: (r, 0, 0)),
            pl.BlockSpec((1, _H, _H), lambda r: (r, 0, 0)),
        ],
        out_specs=pl.BlockSpec((_B, _H), lambda r: (0, 0)),
        out_shape=jax.ShapeDtypeStruct((_B, _H), jnp.float32),
    )(agg, w1, w2)


def kernel(emb, W1, W2, neigh, nodes):
    neighf = neigh.reshape(_R * _N, _S)
    rows1 = jnp.arange(_R, dtype=jnp.int32)[:, None] * _N + nodes[None, :]
    nb2 = neighf[rows1]
    rows2 = (jnp.arange(_R, dtype=jnp.int32)[:, None] * _N
             + nb2.reshape(_R, _B * _S))
    h1 = neighf[rows2]
    idx1 = h1.reshape(_RM * _S)
    agg = _sc_gather_sum(emb, idx1)
    inv_s = jnp.float32(1.0 / _S)
    return _tc_transform(agg.reshape(_R, _M, _F), W1 * inv_s, W2 * inv_s)
